# baseline (speedup 1.0000x reference)
import jax
import jax.numpy as jnp
from jax.experimental import pallas as pl

_INDEX = 1024


def _copy_kernel(x_ref, o_ref):
    o_ref[...] = x_ref[:, _INDEX % 8, :]


def kernel(x):
    b, s, d = x.shape
    return pl.pallas_call(
        _copy_kernel,
        grid=(4,),
        in_specs=[pl.BlockSpec((b, 8, d // 4), lambda i: (0, _INDEX // 8, i))],
        out_specs=pl.BlockSpec((b, d // 4), lambda i: (0, i)),
        out_shape=jax.ShapeDtypeStruct((b, d), x.dtype),
    )(x)


# speedup vs baseline: 1.8056x; 1.8056x over previous
import jax
import jax.numpy as jnp
from jax.experimental import pallas as pl

_INDEX = 1024


def _copy_kernel(x_ref, o_ref):
    o_ref[...] = x_ref[:, _INDEX % 8, :]


def kernel(x):
    b, s, d = x.shape
    return pl.pallas_call(
        _copy_kernel,
        grid=(1,),
        in_specs=[pl.BlockSpec((b, 8, d), lambda i: (0, _INDEX // 8, 0))],
        out_specs=pl.BlockSpec((b, d), lambda i: (0, 0)),
        out_shape=jax.ShapeDtypeStruct((b, d), x.dtype),
    )(x)
